# baseline (speedup 1.0000x reference)
import functools

import jax
import jax.numpy as jnp
from jax import lax
from jax.experimental import pallas as pl
from jax.experimental.pallas import tpu as pltpu
from jax.experimental.pallas import tpu_sc as plsc

N = 10000
E = 320000
D = 128

NC = 2
NS = 16
NW = NC * NS
CHUNK = 128
CPT = 80
EPAD = NW * CPT * CHUNK
NPAD = 10240
RPT = NPAD // NS
HW = 128


def _deg_body(dst_hbm, out_hbm, dst_v, ones_v, acc_sh, sa, sb):
    cid = lax.axis_index("c")
    sid = lax.axis_index("s")
    wid = cid * NS + sid
    pltpu.sync_copy(dst_hbm.at[wid], dst_v)

    def fill(val):
        def body(i, _):
            for j in range(HW // 16):
                ones_v[i, pl.ds(j * 16, 16)] = jnp.full((16,), val, jnp.float32)
            return 0
        lax.fori_loop(0, CHUNK, body, 0)

    fill(0.0)
    base = sid * RPT
    for k in range(RPT // CHUNK):
        pltpu.sync_copy(ones_v, acc_sh.at[pl.ds(base + k * CHUNK, CHUNK)])
    plsc.subcore_barrier()

    fill(1.0)

    def body(t, _):
        d0 = pltpu.async_copy(ones_v, acc_sh.at[dst_v.at[2 * t]], sa,
                              add=True)
        d1 = pltpu.async_copy(ones_v, acc_sh.at[dst_v.at[2 * t + 1]], sb,
                              add=True)
        d0.wait()
        d1.wait()
        return 0
    lax.fori_loop(0, CPT // 2, body, 0)
    plsc.subcore_barrier()
    pltpu.sync_copy(acc_sh.at[pl.ds(base, RPT)],
                    out_hbm.at[cid, pl.ds(base, RPT)])


_PH = 2
_PC = CPT // _PH


def _agg_body(hp_hbm, src_hbm, dst_hbm, out_hbm, src_v, dst_v,
              r0, r1, acc_sh, g0, g1, s0, s1):
    cid = lax.axis_index("c")
    sid = lax.axis_index("s")
    wid = cid * NS + sid

    def zrow(i, _):
        for j in range(D // 16):
            r0[i, pl.ds(j * 16, 16)] = jnp.zeros((16,), jnp.float32)
        return 0
    lax.fori_loop(0, CHUNK, zrow, 0)
    base = sid * RPT
    for k in range(RPT // CHUNK):
        pltpu.sync_copy(r0, acc_sh.at[pl.ds(base + k * CHUNK, CHUNK)])
    plsc.subcore_barrier()

    for ph in range(_PH):
        pltpu.sync_copy(src_hbm.at[wid, pl.ds(ph * _PC, _PC)], src_v)
        pltpu.sync_copy(dst_hbm.at[wid, pl.ds(ph * _PC, _PC)], dst_v)

        def body(t, _):
            gd0 = pltpu.async_copy(hp_hbm.at[src_v.at[2 * t]], r0, g0)
            gd1 = pltpu.async_copy(hp_hbm.at[src_v.at[2 * t + 1]], r1, g1)
            gd0.wait()
            sd0 = pltpu.async_copy(r0, acc_sh.at[dst_v.at[2 * t]], s0,
                                   add=True)
            gd1.wait()
            sd1 = pltpu.async_copy(r1, acc_sh.at[dst_v.at[2 * t + 1]], s1,
                                   add=True)
            sd0.wait()
            sd1.wait()
            return 0
        lax.fori_loop(0, _PC // 2, body, 0)
    plsc.subcore_barrier()
    pltpu.sync_copy(acc_sh.at[pl.ds(base, RPT)],
                    out_hbm.at[cid, pl.ds(base, RPT)])


@functools.cache
def _sc_kernels():
    mesh = plsc.VectorSubcoreMesh(core_axis_name="c", subcore_axis_name="s",
                                  num_cores=NC, num_subcores=NS)
    deg = pl.kernel(
        _deg_body,
        out_type=jax.ShapeDtypeStruct((NC, NPAD, HW), jnp.float32),
        mesh=mesh,
        scratch_types=[
            pltpu.VMEM((CPT, CHUNK), jnp.int32),
            pltpu.VMEM((CHUNK, HW), jnp.float32),
            pltpu.VMEM_SHARED((NPAD, HW), jnp.float32),
            pltpu.SemaphoreType.DMA,
            pltpu.SemaphoreType.DMA,
        ],
    )
    agg = pl.kernel(
        _agg_body,
        out_type=jax.ShapeDtypeStruct((NC, NPAD, D), jnp.float32),
        mesh=mesh,
        scratch_types=[
            pltpu.VMEM((_PC, CHUNK), jnp.int32),
            pltpu.VMEM((_PC, CHUNK), jnp.int32),
            pltpu.VMEM((CHUNK, D), jnp.float32),
            pltpu.VMEM((CHUNK, D), jnp.float32),
            pltpu.VMEM_SHARED((NPAD, D), jnp.float32),
        ] + [pltpu.SemaphoreType.DMA] * 4,
    )
    return deg, agg


_BLK = 256
_GRID = NPAD // _BLK


def _dis(h0_ref, h1_ref):
    deg = h0_ref[...] + h1_ref[...] + 1.0
    return lax.rsqrt(deg)


def _mm_scale_body(h0_ref, h1_ref, x_ref, w_ref, o_ref):
    h = jnp.dot(x_ref[...], w_ref[...], preferred_element_type=jnp.float32)
    o_ref[...] = h * _dis(h0_ref, h1_ref)


def _mid_body(h0_ref, h1_ref, p0_ref, p1_ref, hp_ref, b_ref, w_ref, o_ref):
    dis = _dis(h0_ref, h1_ref)
    z = (p0_ref[...] + p1_ref[...] + hp_ref[...]) * dis + b_ref[...]
    h = jnp.maximum(z, 0.0)
    o_ref[...] = jnp.dot(h, w_ref[...],
                         preferred_element_type=jnp.float32) * dis


def _final_body(h0_ref, h1_ref, p0_ref, p1_ref, hp_ref, b_ref, o_ref):
    dis = _dis(h0_ref, h1_ref)
    z = (p0_ref[...] + p1_ref[...] + hp_ref[...]) * dis + b_ref[...]
    m = jnp.max(z, axis=1, keepdims=True)
    shifted = z - m
    lse = jnp.log(jnp.sum(jnp.exp(shifted), axis=1, keepdims=True))
    o_ref[...] = shifted - lse


def _row_spec(w):
    return pl.BlockSpec((_BLK, w), lambda i: (i, 0))


def _full_spec(r, c):
    return pl.BlockSpec((r, c), lambda i: (0, 0))


_OUT_SPEC = pl.BlockSpec((_BLK, D), lambda i: (i, 0))
_OUT_SHAPE = jax.ShapeDtypeStruct((NPAD, D), jnp.float32)

_tc1 = pl.pallas_call(
    _mm_scale_body,
    grid=(_GRID,),
    in_specs=[_row_spec(HW), _row_spec(HW), _row_spec(D), _full_spec(D, D)],
    out_specs=_OUT_SPEC,
    out_shape=_OUT_SHAPE,
)

_tc2 = pl.pallas_call(
    _mid_body,
    grid=(_GRID,),
    in_specs=[_row_spec(HW), _row_spec(HW), _row_spec(D), _row_spec(D),
              _row_spec(D), _full_spec(1, D), _full_spec(D, D)],
    out_specs=_OUT_SPEC,
    out_shape=_OUT_SHAPE,
)

_tc3 = pl.pallas_call(
    _final_body,
    grid=(_GRID,),
    in_specs=[_row_spec(HW), _row_spec(HW), _row_spec(D), _row_spec(D),
              _row_spec(D), _full_spec(1, D)],
    out_specs=_OUT_SPEC,
    out_shape=_OUT_SHAPE,
)


def kernel(x, edge_index, W1, b1, W2, b2):
    src = edge_index[0]
    dst = edge_index[1]
    pad = EPAD - E
    src_p = jnp.concatenate(
        [src, jnp.zeros((pad,), jnp.int32)]).reshape(NW, CPT, CHUNK)
    dst_p = jnp.concatenate(
        [dst, jnp.full((pad,), NPAD - 1, jnp.int32)]).reshape(NW, CPT, CHUNK)
    xp = jnp.concatenate([x, jnp.zeros((NPAD - N, D), x.dtype)])

    _deg, _agg = _sc_kernels()
    hist = _deg(dst_p)
    h0, h1 = hist[0], hist[1]
    hp1 = _tc1(h0, h1, xp, W1)
    p = _agg(hp1, src_p, dst_p)
    hp2 = _tc2(h0, h1, p[0], p[1], hp1, b1.reshape(1, D), W2)
    p2 = _agg(hp2, src_p, dst_p)
    outp = _tc3(h0, h1, p2[0], p2[1], hp2, b2.reshape(1, D))
    return outp[:N]


# speedup vs baseline: 2.2344x; 2.2344x over previous
import functools

import jax
import jax.numpy as jnp
from jax import lax
from jax.experimental import pallas as pl
from jax.experimental.pallas import tpu as pltpu
from jax.experimental.pallas import tpu_sc as plsc

N = 10000
E = 320000
D = 128

NC = 2
NS = 16
NW = NC * NS
CHUNK = 128
CPT = 80
EPAD = NW * CPT * CHUNK
NPAD = 10240
RPT = NPAD // NS
HW = 128


def _deg_body(dst_hbm, out_hbm, dst_v, ones_v, acc_sh, sa, sb):
    cid = lax.axis_index("c")
    sid = lax.axis_index("s")
    wid = cid * NS + sid
    pltpu.sync_copy(dst_hbm.at[wid], dst_v)

    def fill(val):
        def body(i, _):
            for j in range(HW // 16):
                ones_v[i, pl.ds(j * 16, 16)] = jnp.full((16,), val, jnp.float32)
            return 0
        lax.fori_loop(0, CHUNK, body, 0)

    fill(0.0)
    base = sid * RPT
    for k in range(RPT // CHUNK):
        pltpu.sync_copy(ones_v, acc_sh.at[pl.ds(base + k * CHUNK, CHUNK)])
    plsc.subcore_barrier()

    fill(1.0)

    def body(t, _):
        d0 = pltpu.async_copy(ones_v, acc_sh.at[dst_v.at[2 * t]], sa,
                              add=True)
        d1 = pltpu.async_copy(ones_v, acc_sh.at[dst_v.at[2 * t + 1]], sb,
                              add=True)
        d0.wait()
        d1.wait()
        return 0
    lax.fori_loop(0, CPT // 2, body, 0)
    plsc.subcore_barrier()
    pltpu.sync_copy(acc_sh.at[pl.ds(base, RPT)],
                    out_hbm.at[cid, pl.ds(base, RPT)])


_PH = 2
_PC = CPT // _PH


def _agg_body(hp_hbm, src_hbm, dst_hbm, out_hbm, src_v, dst_v,
              r0, r1, acc_sh, g0, g1, s0, s1):
    cid = lax.axis_index("c")
    sid = lax.axis_index("s")
    wid = cid * NS + sid

    def zrow(i, _):
        for j in range(D // 16):
            r0[i, pl.ds(j * 16, 16)] = jnp.zeros((16,), jnp.float32)
        return 0
    lax.fori_loop(0, CHUNK, zrow, 0)
    base = sid * RPT
    for k in range(RPT // CHUNK):
        pltpu.sync_copy(r0, acc_sh.at[pl.ds(base + k * CHUNK, CHUNK)])
    plsc.subcore_barrier()

    for ph in range(_PH):
        pltpu.sync_copy(src_hbm.at[wid, pl.ds(ph * _PC, _PC)], src_v)
        pltpu.sync_copy(dst_hbm.at[wid, pl.ds(ph * _PC, _PC)], dst_v)

        def body(t, _):
            gd0 = pltpu.async_copy(hp_hbm.at[src_v.at[2 * t]], r0, g0)
            gd1 = pltpu.async_copy(hp_hbm.at[src_v.at[2 * t + 1]], r1, g1)
            gd0.wait()
            sd0 = pltpu.async_copy(r0, acc_sh.at[dst_v.at[2 * t]], s0,
                                   add=True)
            gd1.wait()
            sd1 = pltpu.async_copy(r1, acc_sh.at[dst_v.at[2 * t + 1]], s1,
                                   add=True)
            sd0.wait()
            sd1.wait()
            return 0
        lax.fori_loop(0, _PC // 2, body, 0)
    plsc.subcore_barrier()
    pltpu.sync_copy(acc_sh.at[pl.ds(base, RPT)],
                    out_hbm.at[cid, pl.ds(base, RPT)])


@functools.cache
def _sc_kernels():
    mesh = plsc.VectorSubcoreMesh(core_axis_name="c", subcore_axis_name="s",
                                  num_cores=NC, num_subcores=NS)
    deg = pl.kernel(
        _deg_body,
        out_type=jax.ShapeDtypeStruct((NC, NPAD, HW), jnp.float32),
        mesh=mesh,
        scratch_types=[
            pltpu.VMEM((CPT, CHUNK), jnp.int32),
            pltpu.VMEM((CHUNK, HW), jnp.float32),
            pltpu.VMEM_SHARED((NPAD, HW), jnp.float32),
            pltpu.SemaphoreType.DMA,
            pltpu.SemaphoreType.DMA,
        ],
    )
    agg = pl.kernel(
        _agg_body,
        out_type=jax.ShapeDtypeStruct((NC, NPAD, D), jnp.float32),
        mesh=mesh,
        scratch_types=[
            pltpu.VMEM((_PC, CHUNK), jnp.int32),
            pltpu.VMEM((_PC, CHUNK), jnp.int32),
            pltpu.VMEM((CHUNK, D), jnp.float32),
            pltpu.VMEM((CHUNK, D), jnp.float32),
            pltpu.VMEM_SHARED((NPAD, D), jnp.float32),
        ] + [pltpu.SemaphoreType.DMA] * 4,
    )
    return deg, agg


_BLK = 256
_GRID = NPAD // _BLK


def _dis(h0_ref, h1_ref):
    deg = h0_ref[...] + h1_ref[...] + 1.0
    return lax.rsqrt(deg)


def _mm_scale_body(h0_ref, h1_ref, x_ref, w_ref, o_ref):
    h = jnp.dot(x_ref[...], w_ref[...], preferred_element_type=jnp.float32)
    o_ref[...] = h * _dis(h0_ref, h1_ref)


def _mid_body(h0_ref, h1_ref, p0_ref, p1_ref, hp_ref, b_ref, w_ref, o_ref):
    dis = _dis(h0_ref, h1_ref)
    z = (p0_ref[...] + p1_ref[...] + hp_ref[...]) * dis + b_ref[...]
    h = jnp.maximum(z, 0.0)
    o_ref[...] = jnp.dot(h, w_ref[...],
                         preferred_element_type=jnp.float32) * dis


def _final_body(h0_ref, h1_ref, p0_ref, p1_ref, hp_ref, b_ref, o_ref):
    dis = _dis(h0_ref, h1_ref)
    z = (p0_ref[...] + p1_ref[...] + hp_ref[...]) * dis + b_ref[...]
    m = jnp.max(z, axis=1, keepdims=True)
    shifted = z - m
    lse = jnp.log(jnp.sum(jnp.exp(shifted), axis=1, keepdims=True))
    o_ref[...] = shifted - lse


def _row_spec(w):
    return pl.BlockSpec((_BLK, w), lambda i: (i, 0))


def _full_spec(r, c):
    return pl.BlockSpec((r, c), lambda i: (0, 0))


_OUT_SPEC = pl.BlockSpec((_BLK, D), lambda i: (i, 0))
_OUT_SHAPE = jax.ShapeDtypeStruct((NPAD, D), jnp.float32)

_tc1 = pl.pallas_call(
    _mm_scale_body,
    grid=(_GRID,),
    in_specs=[_row_spec(HW), _row_spec(HW), _row_spec(D), _full_spec(D, D)],
    out_specs=_OUT_SPEC,
    out_shape=_OUT_SHAPE,
)

_tc2 = pl.pallas_call(
    _mid_body,
    grid=(_GRID,),
    in_specs=[_row_spec(HW), _row_spec(HW), _row_spec(D), _row_spec(D),
              _row_spec(D), _full_spec(1, D), _full_spec(D, D)],
    out_specs=_OUT_SPEC,
    out_shape=_OUT_SHAPE,
)

_tc3 = pl.pallas_call(
    _final_body,
    grid=(_GRID,),
    in_specs=[_row_spec(HW), _row_spec(HW), _row_spec(D), _row_spec(D),
              _row_spec(D), _full_spec(1, D)],
    out_specs=_OUT_SPEC,
    out_shape=_OUT_SHAPE,
)


def kernel(x, edge_index, W1, b1, W2, b2):
    src = edge_index[0]
    dst = edge_index[1]
    pad = EPAD - E
    pad_idx = jnp.arange(pad, dtype=jnp.int32)
    src_p = jnp.concatenate(
        [src, pad_idx % N]).reshape(NW, CPT, CHUNK)
    dst_p = jnp.concatenate(
        [dst, N + pad_idx % (NPAD - N)]).reshape(NW, CPT, CHUNK)
    xp = jnp.concatenate([x, jnp.zeros((NPAD - N, D), x.dtype)])

    _deg, _agg = _sc_kernels()
    hist = _deg(dst_p)
    h0, h1 = hist[0], hist[1]
    hp1 = _tc1(h0, h1, xp, W1)
    p = _agg(hp1, src_p, dst_p)
    hp2 = _tc2(h0, h1, p[0], p[1], hp1, b1.reshape(1, D), W2)
    p2 = _agg(hp2, src_p, dst_p)
    outp = _tc3(h0, h1, p2[0], p2[1], hp2, b2.reshape(1, D))
    return outp[:N]


# speedup vs baseline: 2.2416x; 1.0032x over previous
import functools

import jax
import jax.numpy as jnp
from jax import lax
from jax.experimental import pallas as pl
from jax.experimental.pallas import tpu as pltpu
from jax.experimental.pallas import tpu_sc as plsc

N = 10000
E = 320000
D = 128

NC = 2
NS = 16
NW = NC * NS
CHUNK = 128
CPT = 80
EPAD = NW * CPT * CHUNK
NPAD = 10240
RPT = NPAD // NS
HW = 128


def _deg_body(dst_hbm, out_hbm, dst_v, ones_v, acc_sh, sa, sb):
    cid = lax.axis_index("c")
    sid = lax.axis_index("s")
    wid = cid * NS + sid
    pltpu.sync_copy(dst_hbm.at[wid], dst_v)

    def fill(val):
        def body(i, _):
            for j in range(HW // 16):
                ones_v[i, pl.ds(j * 16, 16)] = jnp.full((16,), val, jnp.float32)
            return 0
        lax.fori_loop(0, CHUNK, body, 0)

    fill(0.0)
    base = sid * RPT
    for k in range(RPT // CHUNK):
        pltpu.sync_copy(ones_v, acc_sh.at[pl.ds(base + k * CHUNK, CHUNK)])
    plsc.subcore_barrier()

    fill(1.0)

    def body(t, _):
        d0 = pltpu.async_copy(ones_v, acc_sh.at[dst_v.at[2 * t]], sa,
                              add=True)
        d1 = pltpu.async_copy(ones_v, acc_sh.at[dst_v.at[2 * t + 1]], sb,
                              add=True)
        d0.wait()
        d1.wait()
        return 0
    lax.fori_loop(0, CPT // 2, body, 0)
    plsc.subcore_barrier()
    pltpu.sync_copy(acc_sh.at[pl.ds(base, RPT)],
                    out_hbm.at[cid, pl.ds(base, RPT)])


_PH = 2
_PC = CPT // _PH


def _agg_body(hp_hbm, src_hbm, dst_hbm, out_hbm, src_v, dst_v,
              r0, r1, acc_sh, g0, g1, s0, s1):
    cid = lax.axis_index("c")
    sid = lax.axis_index("s")
    wid = cid * NS + sid

    def zrow(i, _):
        for j in range(D // 16):
            r0[i, pl.ds(j * 16, 16)] = jnp.zeros((16,), jnp.float32)
        return 0
    lax.fori_loop(0, CHUNK, zrow, 0)
    base = sid * RPT
    for k in range(RPT // CHUNK):
        pltpu.sync_copy(r0, acc_sh.at[pl.ds(base + k * CHUNK, CHUNK)])
    plsc.subcore_barrier()

    for ph in range(_PH):
        pltpu.sync_copy(src_hbm.at[wid, pl.ds(ph * _PC, _PC)], src_v)
        pltpu.sync_copy(dst_hbm.at[wid, pl.ds(ph * _PC, _PC)], dst_v)

        def body(t, _):
            gd0 = pltpu.async_copy(hp_hbm.at[src_v.at[2 * t]], r0, g0)
            gd1 = pltpu.async_copy(hp_hbm.at[src_v.at[2 * t + 1]], r1, g1)
            gd0.wait()
            sd0 = pltpu.async_copy(r0, acc_sh.at[dst_v.at[2 * t]], s0,
                                   add=True)
            gd1.wait()
            sd1 = pltpu.async_copy(r1, acc_sh.at[dst_v.at[2 * t + 1]], s1,
                                   add=True)
            sd0.wait()
            sd1.wait()
            return 0
        lax.fori_loop(0, _PC // 2, body, 0)
    plsc.subcore_barrier()
    pltpu.sync_copy(acc_sh.at[pl.ds(base, RPT)],
                    out_hbm.at[cid, pl.ds(base, RPT)])


@functools.cache
def _sc_kernels():
    mesh = plsc.VectorSubcoreMesh(core_axis_name="c", subcore_axis_name="s",
                                  num_cores=NC, num_subcores=NS)
    deg = pl.kernel(
        _deg_body,
        out_type=jax.ShapeDtypeStruct((NC, NPAD, HW), jnp.float32),
        mesh=mesh,
        scratch_types=[
            pltpu.VMEM((CPT, CHUNK), jnp.int32),
            pltpu.VMEM((CHUNK, HW), jnp.float32),
            pltpu.VMEM_SHARED((NPAD, HW), jnp.float32),
            pltpu.SemaphoreType.DMA,
            pltpu.SemaphoreType.DMA,
        ],
    )
    agg = pl.kernel(
        _agg_body,
        out_type=jax.ShapeDtypeStruct((NC, NPAD, D), jnp.float32),
        mesh=mesh,
        scratch_types=[
            pltpu.VMEM((_PC, CHUNK), jnp.int32),
            pltpu.VMEM((_PC, CHUNK), jnp.int32),
            pltpu.VMEM((CHUNK, D), jnp.float32),
            pltpu.VMEM((CHUNK, D), jnp.float32),
            pltpu.VMEM_SHARED((NPAD, D), jnp.float32),
        ] + [pltpu.SemaphoreType.DMA] * 4,
    )
    return deg, agg


_BLK = 256
_GRID = NPAD // _BLK


def _dis(h0_ref, h1_ref):
    deg = h0_ref[...] + h1_ref[...] + 1.0
    return lax.rsqrt(deg)


def _mm_body(x_ref, w_ref, o_ref):
    o_ref[...] = jnp.dot(x_ref[...], w_ref[...],
                         preferred_element_type=jnp.float32)


def _scale_body(h0_ref, h1_ref, h_ref, o_ref):
    o_ref[...] = h_ref[...] * _dis(h0_ref, h1_ref)


def _mid_body(h0_ref, h1_ref, p0_ref, p1_ref, hp_ref, b_ref, w_ref, o_ref):
    dis = _dis(h0_ref, h1_ref)
    z = (p0_ref[...] + p1_ref[...] + hp_ref[...]) * dis + b_ref[...]
    h = jnp.maximum(z, 0.0)
    o_ref[...] = jnp.dot(h, w_ref[...],
                         preferred_element_type=jnp.float32) * dis


def _final_body(h0_ref, h1_ref, p0_ref, p1_ref, hp_ref, b_ref, o_ref):
    dis = _dis(h0_ref, h1_ref)
    z = (p0_ref[...] + p1_ref[...] + hp_ref[...]) * dis + b_ref[...]
    m = jnp.max(z, axis=1, keepdims=True)
    shifted = z - m
    lse = jnp.log(jnp.sum(jnp.exp(shifted), axis=1, keepdims=True))
    o_ref[...] = shifted - lse


def _row_spec(w):
    return pl.BlockSpec((_BLK, w), lambda i: (i, 0))


def _full_spec(r, c):
    return pl.BlockSpec((r, c), lambda i: (0, 0))


_OUT_SPEC = pl.BlockSpec((_BLK, D), lambda i: (i, 0))
_OUT_SHAPE = jax.ShapeDtypeStruct((NPAD, D), jnp.float32)

_tc_mm = pl.pallas_call(
    _mm_body,
    grid=(_GRID,),
    in_specs=[_row_spec(D), _full_spec(D, D)],
    out_specs=_OUT_SPEC,
    out_shape=_OUT_SHAPE,
)

_tc_scale = pl.pallas_call(
    _scale_body,
    grid=(_GRID,),
    in_specs=[_row_spec(HW), _row_spec(HW), _row_spec(D)],
    out_specs=_OUT_SPEC,
    out_shape=_OUT_SHAPE,
)

_tc2 = pl.pallas_call(
    _mid_body,
    grid=(_GRID,),
    in_specs=[_row_spec(HW), _row_spec(HW), _row_spec(D), _row_spec(D),
              _row_spec(D), _full_spec(1, D), _full_spec(D, D)],
    out_specs=_OUT_SPEC,
    out_shape=_OUT_SHAPE,
)

_tc3 = pl.pallas_call(
    _final_body,
    grid=(_GRID,),
    in_specs=[_row_spec(HW), _row_spec(HW), _row_spec(D), _row_spec(D),
              _row_spec(D), _full_spec(1, D)],
    out_specs=_OUT_SPEC,
    out_shape=_OUT_SHAPE,
)


def kernel(x, edge_index, W1, b1, W2, b2):
    src = edge_index[0]
    dst = edge_index[1]
    pad = EPAD - E
    pad_idx = jnp.arange(pad, dtype=jnp.int32)
    src_p = jnp.concatenate(
        [src, pad_idx % N]).reshape(NW, CPT, CHUNK)
    dst_p = jnp.concatenate(
        [dst, N + pad_idx % (NPAD - N)]).reshape(NW, CPT, CHUNK)
    xp = jnp.concatenate([x, jnp.zeros((NPAD - N, D), x.dtype)])

    _deg, _agg = _sc_kernels()
    h1mm = _tc_mm(xp, W1)
    hist = _deg(dst_p)
    h0, h1 = hist[0], hist[1]
    hp1 = _tc_scale(h0, h1, h1mm)
    p = _agg(hp1, src_p, dst_p)
    hp2 = _tc2(h0, h1, p[0], p[1], hp1, b1.reshape(1, D), W2)
    p2 = _agg(hp2, src_p, dst_p)
    outp = _tc3(h0, h1, p2[0], p2[1], hp2, b2.reshape(1, D))
    return outp[:N]


# speedup vs baseline: 2.2689x; 1.0122x over previous
import functools

import jax
import jax.numpy as jnp
from jax import lax
from jax.experimental import pallas as pl
from jax.experimental.pallas import tpu as pltpu
from jax.experimental.pallas import tpu_sc as plsc

N = 10000
E = 320000
D = 128

NC = 2
NS = 16
NW = NC * NS
CHUNK = 128
CPT = 80
EPAD = NW * CPT * CHUNK
NPAD = 10240
RPT = NPAD // NS
HW = 128


def _deg_body(dst_hbm, out_hbm, dst_v, ones_v, acc_sh, sa, sb):
    cid = lax.axis_index("c")
    sid = lax.axis_index("s")
    wid = cid * NS + sid
    pltpu.sync_copy(dst_hbm.at[wid], dst_v)

    def fill(val):
        def body(i, _):
            for j in range(HW // 16):
                ones_v[i, pl.ds(j * 16, 16)] = jnp.full((16,), val, jnp.float32)
            return 0
        lax.fori_loop(0, CHUNK, body, 0)

    fill(0.0)
    base = sid * RPT
    for k in range(RPT // CHUNK):
        pltpu.sync_copy(ones_v, acc_sh.at[pl.ds(base + k * CHUNK, CHUNK)])
    plsc.subcore_barrier()

    fill(1.0)

    def body(t, _):
        d0 = pltpu.async_copy(ones_v, acc_sh.at[dst_v.at[2 * t]], sa,
                              add=True)
        d1 = pltpu.async_copy(ones_v, acc_sh.at[dst_v.at[2 * t + 1]], sb,
                              add=True)
        d0.wait()
        d1.wait()
        return 0
    lax.fori_loop(0, CPT // 2, body, 0)
    plsc.subcore_barrier()
    pltpu.sync_copy(acc_sh.at[pl.ds(base, RPT)],
                    out_hbm.at[cid, pl.ds(base, RPT)])


_PH = 2
_PC = CPT // _PH


def _agg_body(hp_hbm, src_hbm, dst_hbm, out_hbm, src_v, dst_v,
              r0, r1, acc_sh, g0, g1, s0, s1):
    cid = lax.axis_index("c")
    sid = lax.axis_index("s")
    wid = cid * NS + sid

    def zrow(i, _):
        for j in range(D // 16):
            r0[i, pl.ds(j * 16, 16)] = jnp.zeros((16,), jnp.float32)
        return 0
    lax.fori_loop(0, CHUNK, zrow, 0)
    base = sid * RPT
    for k in range(RPT // CHUNK):
        pltpu.sync_copy(r0, acc_sh.at[pl.ds(base + k * CHUNK, CHUNK)])
    plsc.subcore_barrier()

    def gissue(c, buf, sem):
        pltpu.async_copy(hp_hbm.at[src_v.at[c]], buf, sem)

    def sissue(c, buf, sem):
        pltpu.async_copy(buf, acc_sh.at[dst_v.at[c]], sem, add=True)

    def gwait(buf, sem):
        pltpu.make_async_copy(hp_hbm.at[src_v.at[0]], buf, sem).wait()

    def swait(buf, sem):
        pltpu.make_async_copy(buf, acc_sh.at[dst_v.at[0]], sem).wait()

    for ph in range(_PH):
        pltpu.sync_copy(src_hbm.at[wid, pl.ds(ph * _PC, _PC)], src_v)
        pltpu.sync_copy(dst_hbm.at[wid, pl.ds(ph * _PC, _PC)], dst_v)
        gissue(0, r0, g0)
        gissue(1, r1, g1)
        gwait(r0, g0)
        sissue(0, r0, s0)
        gwait(r1, g1)
        sissue(1, r1, s1)

        def body(t, _):
            swait(r0, s0)
            gissue(2 * t, r0, g0)
            swait(r1, s1)
            gissue(2 * t + 1, r1, g1)
            gwait(r0, g0)
            sissue(2 * t, r0, s0)
            gwait(r1, g1)
            sissue(2 * t + 1, r1, s1)
            return 0
        lax.fori_loop(1, _PC // 2, body, 0)
        swait(r0, s0)
        swait(r1, s1)
    plsc.subcore_barrier()
    pltpu.sync_copy(acc_sh.at[pl.ds(base, RPT)],
                    out_hbm.at[cid, pl.ds(base, RPT)])


@functools.cache
def _sc_kernels():
    mesh = plsc.VectorSubcoreMesh(core_axis_name="c", subcore_axis_name="s",
                                  num_cores=NC, num_subcores=NS)
    deg = pl.kernel(
        _deg_body,
        out_type=jax.ShapeDtypeStruct((NC, NPAD, HW), jnp.float32),
        mesh=mesh,
        scratch_types=[
            pltpu.VMEM((CPT, CHUNK), jnp.int32),
            pltpu.VMEM((CHUNK, HW), jnp.float32),
            pltpu.VMEM_SHARED((NPAD, HW), jnp.float32),
            pltpu.SemaphoreType.DMA,
            pltpu.SemaphoreType.DMA,
        ],
    )
    agg = pl.kernel(
        _agg_body,
        out_type=jax.ShapeDtypeStruct((NC, NPAD, D), jnp.float32),
        mesh=mesh,
        scratch_types=[
            pltpu.VMEM((_PC, CHUNK), jnp.int32),
            pltpu.VMEM((_PC, CHUNK), jnp.int32),
            pltpu.VMEM((CHUNK, D), jnp.float32),
            pltpu.VMEM((CHUNK, D), jnp.float32),
            pltpu.VMEM_SHARED((NPAD, D), jnp.float32),
        ] + [pltpu.SemaphoreType.DMA] * 4,
    )
    return deg, agg


_BLK = 256
_GRID = NPAD // _BLK


def _dis(h0_ref, h1_ref):
    deg = h0_ref[...] + h1_ref[...] + 1.0
    return lax.rsqrt(deg)


def _mm_body(x_ref, w_ref, o_ref):
    o_ref[...] = jnp.dot(x_ref[...], w_ref[...],
                         preferred_element_type=jnp.float32)


def _scale_body(h0_ref, h1_ref, h_ref, o_ref):
    o_ref[...] = h_ref[...] * _dis(h0_ref, h1_ref)


def _mid_body(h0_ref, h1_ref, p0_ref, p1_ref, hp_ref, b_ref, w_ref, o_ref):
    dis = _dis(h0_ref, h1_ref)
    z = (p0_ref[...] + p1_ref[...] + hp_ref[...]) * dis + b_ref[...]
    h = jnp.maximum(z, 0.0)
    o_ref[...] = jnp.dot(h, w_ref[...],
                         preferred_element_type=jnp.float32) * dis


def _final_body(h0_ref, h1_ref, p0_ref, p1_ref, hp_ref, b_ref, o_ref):
    dis = _dis(h0_ref, h1_ref)
    z = (p0_ref[...] + p1_ref[...] + hp_ref[...]) * dis + b_ref[...]
    m = jnp.max(z, axis=1, keepdims=True)
    shifted = z - m
    lse = jnp.log(jnp.sum(jnp.exp(shifted), axis=1, keepdims=True))
    o_ref[...] = shifted - lse


def _row_spec(w):
    return pl.BlockSpec((_BLK, w), lambda i: (i, 0))


def _full_spec(r, c):
    return pl.BlockSpec((r, c), lambda i: (0, 0))


_OUT_SPEC = pl.BlockSpec((_BLK, D), lambda i: (i, 0))
_OUT_SHAPE = jax.ShapeDtypeStruct((NPAD, D), jnp.float32)

_tc_mm = pl.pallas_call(
    _mm_body,
    grid=(_GRID,),
    in_specs=[_row_spec(D), _full_spec(D, D)],
    out_specs=_OUT_SPEC,
    out_shape=_OUT_SHAPE,
)

_tc_scale = pl.pallas_call(
    _scale_body,
    grid=(_GRID,),
    in_specs=[_row_spec(HW), _row_spec(HW), _row_spec(D)],
    out_specs=_OUT_SPEC,
    out_shape=_OUT_SHAPE,
)

_tc2 = pl.pallas_call(
    _mid_body,
    grid=(_GRID,),
    in_specs=[_row_spec(HW), _row_spec(HW), _row_spec(D), _row_spec(D),
              _row_spec(D), _full_spec(1, D), _full_spec(D, D)],
    out_specs=_OUT_SPEC,
    out_shape=_OUT_SHAPE,
)

_tc3 = pl.pallas_call(
    _final_body,
    grid=(_GRID,),
    in_specs=[_row_spec(HW), _row_spec(HW), _row_spec(D), _row_spec(D),
              _row_spec(D), _full_spec(1, D)],
    out_specs=_OUT_SPEC,
    out_shape=_OUT_SHAPE,
)


def kernel(x, edge_index, W1, b1, W2, b2):
    src = edge_index[0]
    dst = edge_index[1]
    pad = EPAD - E
    pad_idx = jnp.arange(pad, dtype=jnp.int32)
    src_p = jnp.concatenate(
        [src, pad_idx % N]).reshape(NW, CPT, CHUNK)
    dst_p = jnp.concatenate(
        [dst, N + pad_idx % (NPAD - N)]).reshape(NW, CPT, CHUNK)
    xp = jnp.concatenate([x, jnp.zeros((NPAD - N, D), x.dtype)])

    _deg, _agg = _sc_kernels()
    h1mm = _tc_mm(xp, W1)
    hist = _deg(dst_p)
    h0, h1 = hist[0], hist[1]
    hp1 = _tc_scale(h0, h1, h1mm)
    p = _agg(hp1, src_p, dst_p)
    hp2 = _tc2(h0, h1, p[0], p[1], hp1, b1.reshape(1, D), W2)
    p2 = _agg(hp2, src_p, dst_p)
    outp = _tc3(h0, h1, p2[0], p2[1], hp2, b2.reshape(1, D))
    return outp[:N]
